# baseline (speedup 1.0000x reference)
import functools

import jax
import jax.numpy as jnp
from jax import lax
from jax.experimental import pallas as pl
from jax.experimental.pallas import tpu as pltpu
from jax.experimental.pallas import tpu_sc as plsc

NC = 2
NS = 16
NW = NC * NS
CH = 128
LANES = 16

F32 = jnp.float32


def _fill(ref, b, rows, cols, val):
    v16 = jnp.full((LANES,), val, F32)
    groups = cols // LANES

    def body(i, c):
        r = i // groups
        g = i - r * groups
        ref[b, r, pl.ds(g * LANES, LANES)] = v16
        return c

    lax.fori_loop(0, rows * groups, body, 0)


def _make_sc_deg(npad, nch):
    zr = npad // NS
    nch2 = nch // 2
    mesh = plsc.VectorSubcoreMesh(core_axis_name="c", subcore_axis_name="s")

    @functools.partial(
        pl.kernel,
        out_type=jax.ShapeDtypeStruct((NC, npad, LANES), F32),
        mesh=mesh,
        scratch_types=[
            pltpu.VMEM((nch2, CH), jnp.int32),
            pltpu.VMEM((1, CH, LANES), F32),
            pltpu.VMEM_SHARED((npad, LANES), F32),
        ],
        compiler_params=pltpu.CompilerParams(use_tc_tiling_on_sc=False),
    )
    def degk(dst_hbm, out_hbm, didx, ones_rows, acc):
        cid = lax.axis_index("c")
        sid = lax.axis_index("s")
        pltpu.sync_copy(dst_hbm.at[sid, pl.ds(cid * nch2, nch2)], didx)
        _fill(ones_rows, 0, CH, LANES, 0.0)
        for k in range(zr // CH):
            pltpu.sync_copy(ones_rows.at[0],
                            acc.at[pl.ds(sid * zr + k * CH, CH)])
        _fill(ones_rows, 0, CH, LANES, 1.0)
        plsc.subcore_barrier()

        def body(j, c):
            pltpu.sync_copy(ones_rows.at[0], acc.at[didx.at[j]], add=True)
            return c

        lax.fori_loop(0, nch2, body, 0)
        plsc.subcore_barrier()
        for k in range(zr // CH):
            s = sid * zr + k * CH
            pltpu.sync_copy(acc.at[pl.ds(s, CH)], out_hbm.at[cid, pl.ds(s, CH)])

    return degk


def _make_sc_agg(npad, nch, fh, edge_split):
    zr = npad // NS
    NB = 8
    FD = 4
    ncht = nch // 2 if edge_split else nch
    assert ncht % NB == 0 and ncht >= 2 * NB
    mesh = plsc.VectorSubcoreMesh(core_axis_name="c", subcore_axis_name="s")

    @functools.partial(
        pl.kernel,
        out_type=jax.ShapeDtypeStruct((NC, npad, fh), F32),
        mesh=mesh,
        scratch_types=[
            pltpu.VMEM((ncht, CH), jnp.int32),
            pltpu.VMEM((ncht, CH), jnp.int32),
            pltpu.VMEM((NB, CH, fh), F32),
            pltpu.VMEM_SHARED((npad, fh), F32),
            [pltpu.SemaphoreType.DMA] * NB,
            [pltpu.SemaphoreType.DMA] * NB,
        ],
        compiler_params=pltpu.CompilerParams(use_tc_tiling_on_sc=False),
    )
    def aggk(g_hbm, src_hbm, dst_hbm, out_hbm, sidx, didx, rows, acc,
             gsem, ssem):
        cid = lax.axis_index("c")
        sid = lax.axis_index("s")
        if edge_split:
            gtab = g_hbm
            pltpu.sync_copy(src_hbm.at[sid, pl.ds(cid * ncht, ncht)], sidx)
            pltpu.sync_copy(dst_hbm.at[sid, pl.ds(cid * ncht, ncht)], didx)
        else:
            gtab = g_hbm.at[cid]
            pltpu.sync_copy(src_hbm.at[sid], sidx)
            pltpu.sync_copy(dst_hbm.at[sid], didx)
        _fill(rows, 0, CH, fh, 0.0)
        for k in range(zr // CH):
            pltpu.sync_copy(rows.at[0], acc.at[pl.ds(sid * zr + k * CH, CH)])
        plsc.subcore_barrier()

        def visit(v, b, swait, fire):
            pltpu.make_async_copy(gtab.at[sidx.at[v]], rows.at[b],
                                  gsem[b]).wait()
            pltpu.async_copy(rows.at[b], acc.at[didx.at[v]], ssem[b],
                             add=True)
            c = (b + FD) % NB
            if swait:
                pltpu.make_async_copy(rows.at[c], acc.at[didx.at[v + FD - NB]],
                                      ssem[c]).wait()
            if fire:
                pltpu.async_copy(gtab.at[sidx.at[v + FD]], rows.at[c],
                                 gsem[c])

        for b in range(FD):
            pltpu.async_copy(gtab.at[sidx.at[b]], rows.at[b], gsem[b])
        for b in range(NB):
            visit(b, b, swait=(b >= NB - FD), fire=True)

        def body(jj, c):
            for b in range(NB):
                visit(jj * NB + b, b, swait=True, fire=True)
            return c

        lax.fori_loop(1, ncht // NB - 1, body, 0)
        for b in range(NB):
            v = ncht - NB + b
            visit(v, b, swait=(v + FD < ncht), fire=(v + FD < ncht))
        for b in range(NB):
            pltpu.make_async_copy(rows.at[b], acc.at[didx.at[ncht - NB + b]],
                                  ssem[b]).wait()

        plsc.subcore_barrier()
        for k in range(zr // CH):
            s = sid * zr + k * CH
            pltpu.sync_copy(acc.at[pl.ds(s, CH)], out_hbm.at[cid, pl.ds(s, CH)])

    return aggk


def _tc_first(x, W1, degp, n_real, npad, blk):
    fin, f1 = W1.shape
    fh = f1 // 2

    def body(d0_ref, d1_ref, x_ref, w_ref, o_ref):
        deg = d0_ref[0, :, 0:1] + d1_ref[0, :, 0:1] + 1.0
        dinv = lax.rsqrt(deg)
        h = jnp.dot(x_ref[...], w_ref[...], preferred_element_type=F32)
        g = h * dinv
        o_ref[0] = g[:, :fh]
        o_ref[1] = g[:, fh:]

    return pl.pallas_call(
        body,
        grid=(npad // blk,),
        in_specs=[
            pl.BlockSpec((1, blk, LANES), lambda i: (0, i, 0)),
            pl.BlockSpec((1, blk, LANES), lambda i: (1, i, 0)),
            pl.BlockSpec((blk, fin), lambda i: (i, 0)),
            pl.BlockSpec((fin, f1), lambda i: (0, 0)),
        ],
        out_specs=pl.BlockSpec((NC, blk, fh), lambda i: (0, i, 0)),
        out_shape=jax.ShapeDtypeStruct((NC, npad, fh), F32),
    )(degp, degp, x, W1)


def _tc_mid(s1, g1, degp, W2, b1, n_real, npad, blk):
    f1, f2 = W2.shape
    fh1 = f1 // 2

    def body(d0_ref, d1_ref, s_ref, g_ref, w_ref, b_ref, o_ref):
        i = pl.program_id(0)
        deg = d0_ref[0, :, 0:1] + d1_ref[0, :, 0:1] + 1.0
        dinv = lax.rsqrt(deg)
        tot = jnp.concatenate([s_ref[0] + g_ref[0], s_ref[1] + g_ref[1]],
                              axis=1)
        u = jnp.maximum(tot * dinv + b_ref[...], 0.0)
        rows = i * blk + lax.broadcasted_iota(jnp.int32, (blk, 1), 0)
        u = jnp.where(rows < n_real, u, 0.0)
        o_ref[...] = jnp.dot(u, w_ref[...], preferred_element_type=F32) * dinv

    return pl.pallas_call(
        body,
        grid=(npad // blk,),
        in_specs=[
            pl.BlockSpec((1, blk, LANES), lambda i: (0, i, 0)),
            pl.BlockSpec((1, blk, LANES), lambda i: (1, i, 0)),
            pl.BlockSpec((NC, blk, fh1), lambda i: (0, i, 0)),
            pl.BlockSpec((NC, blk, fh1), lambda i: (0, i, 0)),
            pl.BlockSpec((f1, f2), lambda i: (0, 0)),
            pl.BlockSpec((1, f1), lambda i: (0, 0)),
        ],
        out_specs=pl.BlockSpec((blk, f2), lambda i: (i, 0)),
        out_shape=jax.ShapeDtypeStruct((npad, f2), F32),
    )(degp, degp, s1, g1, W2, b1.reshape(1, f1))


def _tc_last(s2, g2, degp, b2, npad, blk):
    f2 = g2.shape[1]

    def body(d0_ref, d1_ref, s0_ref, s1_ref, g_ref, b_ref, o_ref):
        deg = d0_ref[0, :, 0:1] + d1_ref[0, :, 0:1] + 1.0
        dinv = lax.rsqrt(deg)
        tot = s0_ref[0] + s1_ref[0] + g_ref[...]
        o_ref[...] = tot * dinv + b_ref[...]

    return pl.pallas_call(
        body,
        grid=(npad // blk,),
        in_specs=[
            pl.BlockSpec((1, blk, LANES), lambda i: (0, i, 0)),
            pl.BlockSpec((1, blk, LANES), lambda i: (1, i, 0)),
            pl.BlockSpec((1, blk, f2), lambda i: (0, i, 0)),
            pl.BlockSpec((1, blk, f2), lambda i: (1, i, 0)),
            pl.BlockSpec((blk, f2), lambda i: (i, 0)),
            pl.BlockSpec((1, f2), lambda i: (0, 0)),
        ],
        out_specs=pl.BlockSpec((blk, f2), lambda i: (i, 0)),
        out_shape=jax.ShapeDtypeStruct((npad, f2), F32),
    )(degp, degp, s2, s2, g2, b2.reshape(1, f2))


def kernel(x, edge_index, W1, b1, W2, b2):
    n, fin = x.shape
    e = edge_index.shape[1]
    f1 = W1.shape[1]
    f2 = W2.shape[1]

    npad = -((n + 1) // -(NS * CH)) * (NS * CH)
    ep = -(e // -(NS * CH * 8)) * (NS * CH * 8)
    nch = ep // (NS * CH)

    xp = jnp.zeros((npad, fin), F32).at[:n].set(x)
    pad = jnp.full((ep - e,), n, jnp.int32)
    srcp = jnp.concatenate([edge_index[0], pad]).reshape(NS, nch, CH)
    dstp = jnp.concatenate([edge_index[1], pad]).reshape(NS, nch, CH)

    degp = _make_sc_deg(npad, nch)(dstp)
    g1 = _tc_first(xp, W1, degp, n, npad, 1024)
    s1 = _make_sc_agg(npad, nch, f1 // 2, False)(g1, srcp, dstp)
    g2 = _tc_mid(s1, g1, degp, W2, b1, n, npad, 1024)
    s2 = _make_sc_agg(npad, nch, f2, True)(g2, srcp, dstp)
    z = _tc_last(s2, g2, degp, b2, npad, 1024)
    return z[:n]


# speedup vs baseline: 1.0554x; 1.0554x over previous
import functools

import jax
import jax.numpy as jnp
from jax import lax
from jax.experimental import pallas as pl
from jax.experimental.pallas import tpu as pltpu
from jax.experimental.pallas import tpu_sc as plsc

NC = 2
NS = 16
NW = NC * NS
CH = 128
LANES = 16

F32 = jnp.float32


def _fill(ref, b, rows, cols, val):
    v16 = jnp.full((LANES,), val, F32)
    groups = cols // LANES

    def body(i, c):
        r = i // groups
        g = i - r * groups
        ref[b, r, pl.ds(g * LANES, LANES)] = v16
        return c

    lax.fori_loop(0, rows * groups, body, 0)


def _make_sc_deg(npad, nch):
    zr = npad // NS
    nch2 = nch // 2
    mesh = plsc.VectorSubcoreMesh(core_axis_name="c", subcore_axis_name="s")

    @functools.partial(
        pl.kernel,
        out_type=jax.ShapeDtypeStruct((NC, npad, LANES), F32),
        mesh=mesh,
        scratch_types=[
            pltpu.VMEM((nch2, CH), jnp.int32),
            pltpu.VMEM((1, CH, LANES), F32),
            pltpu.VMEM_SHARED((npad, LANES), F32),
        ],
        compiler_params=pltpu.CompilerParams(use_tc_tiling_on_sc=False),
    )
    def degk(dst_hbm, out_hbm, didx, ones_rows, acc):
        cid = lax.axis_index("c")
        sid = lax.axis_index("s")
        pltpu.sync_copy(dst_hbm.at[sid, pl.ds(cid * nch2, nch2)], didx)
        _fill(ones_rows, 0, CH, LANES, 0.0)
        for k in range(zr // CH):
            pltpu.sync_copy(ones_rows.at[0],
                            acc.at[pl.ds(sid * zr + k * CH, CH)])
        _fill(ones_rows, 0, CH, LANES, 1.0)
        plsc.subcore_barrier()

        def body(j, c):
            pltpu.sync_copy(ones_rows.at[0], acc.at[didx.at[j]], add=True)
            return c

        lax.fori_loop(0, nch2, body, 0)
        plsc.subcore_barrier()
        for k in range(zr // CH):
            s = sid * zr + k * CH
            pltpu.sync_copy(acc.at[pl.ds(s, CH)], out_hbm.at[cid, pl.ds(s, CH)])

    return degk


def _make_sc_agg(npad, nch, fh, edge_split):
    zr = npad // NS
    NB = 8
    FD = 4
    ncht = nch // 2 if edge_split else nch
    assert ncht % NB == 0 and ncht >= 2 * NB
    mesh = plsc.VectorSubcoreMesh(core_axis_name="c", subcore_axis_name="s")

    @functools.partial(
        pl.kernel,
        out_type=jax.ShapeDtypeStruct((NC, npad, fh), F32),
        mesh=mesh,
        scratch_types=[
            pltpu.VMEM((ncht, CH), jnp.int32),
            pltpu.VMEM((ncht, CH), jnp.int32),
            pltpu.VMEM((NB, CH, fh), F32),
            pltpu.VMEM_SHARED((npad, fh), F32),
            [pltpu.SemaphoreType.DMA] * NB,
            [pltpu.SemaphoreType.DMA] * NB,
        ],
        compiler_params=pltpu.CompilerParams(use_tc_tiling_on_sc=False),
    )
    def aggk(g_hbm, src_hbm, dst_hbm, out_hbm, sidx, didx, rows, acc,
             gsem, ssem):
        cid = lax.axis_index("c")
        sid = lax.axis_index("s")
        if edge_split:
            gtab = g_hbm
            pltpu.sync_copy(src_hbm.at[sid, pl.ds(cid * ncht, ncht)], sidx)
            pltpu.sync_copy(dst_hbm.at[sid, pl.ds(cid * ncht, ncht)], didx)
        else:
            gtab = g_hbm.at[cid]
            pltpu.sync_copy(src_hbm.at[sid], sidx)
            pltpu.sync_copy(dst_hbm.at[sid], didx)
        _fill(rows, 0, CH, fh, 0.0)
        for k in range(zr // CH):
            pltpu.sync_copy(rows.at[0], acc.at[pl.ds(sid * zr + k * CH, CH)])
        plsc.subcore_barrier()

        def visit(v, b, swait, fire):
            pltpu.make_async_copy(gtab.at[sidx.at[v]], rows.at[b],
                                  gsem[b]).wait()
            pltpu.async_copy(rows.at[b], acc.at[didx.at[v]], ssem[b],
                             add=True)
            c = (b + FD) % NB
            if swait:
                pltpu.make_async_copy(rows.at[c], acc.at[didx.at[v + FD - NB]],
                                      ssem[c]).wait()
            if fire:
                pltpu.async_copy(gtab.at[sidx.at[v + FD]], rows.at[c],
                                 gsem[c])

        for b in range(FD):
            pltpu.async_copy(gtab.at[sidx.at[b]], rows.at[b], gsem[b])
        for b in range(NB):
            visit(b, b, swait=(b >= NB - FD), fire=True)

        def body(jj, c):
            for b in range(NB):
                visit(jj * NB + b, b, swait=True, fire=True)
            return c

        lax.fori_loop(1, ncht // NB - 1, body, 0)
        for b in range(NB):
            v = ncht - NB + b
            visit(v, b, swait=(v + FD < ncht), fire=(v + FD < ncht))
        for b in range(NB):
            pltpu.make_async_copy(rows.at[b], acc.at[didx.at[ncht - NB + b]],
                                  ssem[b]).wait()

        plsc.subcore_barrier()
        for k in range(zr // CH):
            s = sid * zr + k * CH
            pltpu.sync_copy(acc.at[pl.ds(s, CH)], out_hbm.at[cid, pl.ds(s, CH)])

    return aggk


def _tc_first(x, W1, degp, n_real, npad, blk):
    fin, f1 = W1.shape
    fh = f1 // 2

    def body(d0_ref, d1_ref, x_ref, w_ref, o_ref):
        deg = d0_ref[0, :, 0:1] + d1_ref[0, :, 0:1] + 1.0
        dinv = lax.rsqrt(deg)
        h = jnp.dot(x_ref[...], w_ref[...], preferred_element_type=F32)
        g = h * dinv
        o_ref[0] = g[:, :fh]
        o_ref[1] = g[:, fh:]

    return pl.pallas_call(
        body,
        grid=(npad // blk,),
        in_specs=[
            pl.BlockSpec((1, blk, LANES), lambda i: (0, i, 0)),
            pl.BlockSpec((1, blk, LANES), lambda i: (1, i, 0)),
            pl.BlockSpec((blk, fin), lambda i: (i, 0)),
            pl.BlockSpec((fin, f1), lambda i: (0, 0)),
        ],
        out_specs=pl.BlockSpec((NC, blk, fh), lambda i: (0, i, 0)),
        out_shape=jax.ShapeDtypeStruct((NC, npad, fh), F32),
    )(degp, degp, x, W1)


def _tc_mid(s1, g1, degp, W2, b1, n_real, npad, blk):
    f1, f2 = W2.shape
    fh1 = f1 // 2
    fh2 = f2 // 2

    def body(d0_ref, d1_ref, s_ref, g_ref, w_ref, b_ref, o_ref):
        i = pl.program_id(0)
        deg = d0_ref[0, :, 0:1] + d1_ref[0, :, 0:1] + 1.0
        dinv = lax.rsqrt(deg)
        tot = jnp.concatenate([s_ref[0] + g_ref[0], s_ref[1] + g_ref[1]],
                              axis=1)
        u = jnp.maximum(tot * dinv + b_ref[...], 0.0)
        rows = i * blk + lax.broadcasted_iota(jnp.int32, (blk, 1), 0)
        u = jnp.where(rows < n_real, u, 0.0)
        g = jnp.dot(u, w_ref[...], preferred_element_type=F32) * dinv
        o_ref[0] = g[:, :fh2]
        o_ref[1] = g[:, fh2:]

    return pl.pallas_call(
        body,
        grid=(npad // blk,),
        in_specs=[
            pl.BlockSpec((1, blk, LANES), lambda i: (0, i, 0)),
            pl.BlockSpec((1, blk, LANES), lambda i: (1, i, 0)),
            pl.BlockSpec((NC, blk, fh1), lambda i: (0, i, 0)),
            pl.BlockSpec((NC, blk, fh1), lambda i: (0, i, 0)),
            pl.BlockSpec((f1, f2), lambda i: (0, 0)),
            pl.BlockSpec((1, f1), lambda i: (0, 0)),
        ],
        out_specs=pl.BlockSpec((NC, blk, fh2), lambda i: (0, i, 0)),
        out_shape=jax.ShapeDtypeStruct((NC, npad, fh2), F32),
    )(degp, degp, s1, g1, W2, b1.reshape(1, f1))


def _tc_last(s2, g2, degp, b2, npad, blk):
    f2 = 2 * g2.shape[2]
    fh2 = f2 // 2

    def body(d0_ref, d1_ref, s_ref, g_ref, b_ref, o_ref):
        deg = d0_ref[0, :, 0:1] + d1_ref[0, :, 0:1] + 1.0
        dinv = lax.rsqrt(deg)
        tot = jnp.concatenate([s_ref[0] + g_ref[0], s_ref[1] + g_ref[1]],
                              axis=1)
        o_ref[...] = tot * dinv + b_ref[...]

    return pl.pallas_call(
        body,
        grid=(npad // blk,),
        in_specs=[
            pl.BlockSpec((1, blk, LANES), lambda i: (0, i, 0)),
            pl.BlockSpec((1, blk, LANES), lambda i: (1, i, 0)),
            pl.BlockSpec((NC, blk, fh2), lambda i: (0, i, 0)),
            pl.BlockSpec((NC, blk, fh2), lambda i: (0, i, 0)),
            pl.BlockSpec((1, f2), lambda i: (0, 0)),
        ],
        out_specs=pl.BlockSpec((blk, f2), lambda i: (i, 0)),
        out_shape=jax.ShapeDtypeStruct((npad, f2), F32),
    )(degp, degp, s2, g2, b2.reshape(1, f2))


def kernel(x, edge_index, W1, b1, W2, b2):
    n, fin = x.shape
    e = edge_index.shape[1]
    f1 = W1.shape[1]
    f2 = W2.shape[1]

    npad = -((n + 1) // -(NS * CH)) * (NS * CH)
    ep = -(e // -(NS * CH * 8)) * (NS * CH * 8)
    nch = ep // (NS * CH)

    xp = jnp.zeros((npad, fin), F32).at[:n].set(x)
    pad = jnp.full((ep - e,), n, jnp.int32)
    srcp = jnp.concatenate([edge_index[0], pad]).reshape(NS, nch, CH)
    dstp = jnp.concatenate([edge_index[1], pad]).reshape(NS, nch, CH)

    degp = _make_sc_deg(npad, nch)(dstp)
    g1 = _tc_first(xp, W1, degp, n, npad, 1024)
    s1 = _make_sc_agg(npad, nch, f1 // 2, False)(g1, srcp, dstp)
    g2 = _tc_mid(s1, g1, degp, W2, b1, n, npad, 1024)
    s2 = _make_sc_agg(npad, nch, f2 // 2, False)(g2, srcp, dstp)
    z = _tc_last(s2, g2, degp, b2, npad, 1024)
    return z[:n]


# speedup vs baseline: 1.0637x; 1.0078x over previous
import functools

import jax
import jax.numpy as jnp
from jax import lax
from jax.experimental import pallas as pl
from jax.experimental.pallas import tpu as pltpu
from jax.experimental.pallas import tpu_sc as plsc

NC = 2
NS = 16
NW = NC * NS
CH = 128
LANES = 16

F32 = jnp.float32


def _fill(ref, b, rows, cols, val):
    v16 = jnp.full((LANES,), val, F32)
    groups = cols // LANES

    def body(i, c):
        r = i // groups
        g = i - r * groups
        ref[b, r, pl.ds(g * LANES, LANES)] = v16
        return c

    lax.fori_loop(0, rows * groups, body, 0)


def _make_sc_deg(npad, nch):
    zr = npad // NS
    nch2 = nch // 2
    mesh = plsc.VectorSubcoreMesh(core_axis_name="c", subcore_axis_name="s")

    @functools.partial(
        pl.kernel,
        out_type=jax.ShapeDtypeStruct((NC, npad, LANES), F32),
        mesh=mesh,
        scratch_types=[
            pltpu.VMEM((nch2, CH), jnp.int32),
            pltpu.VMEM((1, CH, LANES), F32),
            pltpu.VMEM_SHARED((npad, LANES), F32),
        ],
        compiler_params=pltpu.CompilerParams(use_tc_tiling_on_sc=False),
    )
    def degk(dst_hbm, out_hbm, didx, ones_rows, acc):
        cid = lax.axis_index("c")
        sid = lax.axis_index("s")
        pltpu.sync_copy(dst_hbm.at[sid, pl.ds(cid * nch2, nch2)], didx)
        _fill(ones_rows, 0, CH, LANES, 0.0)
        for k in range(zr // CH):
            pltpu.sync_copy(ones_rows.at[0],
                            acc.at[pl.ds(sid * zr + k * CH, CH)])
        _fill(ones_rows, 0, CH, LANES, 1.0)
        plsc.subcore_barrier()

        def body(j, c):
            pltpu.sync_copy(ones_rows.at[0], acc.at[didx.at[j]], add=True)
            return c

        lax.fori_loop(0, nch2, body, 0)
        plsc.subcore_barrier()
        for k in range(zr // CH):
            s = sid * zr + k * CH
            pltpu.sync_copy(acc.at[pl.ds(s, CH)], out_hbm.at[cid, pl.ds(s, CH)])

    return degk


def _make_sc_agg(npad, nch, fh, edge_split):
    zr = npad // NS
    NB = 8
    FD = 4
    ncht = nch // 2 if edge_split else nch
    assert ncht % NB == 0 and ncht >= 2 * NB
    mesh = plsc.VectorSubcoreMesh(core_axis_name="c", subcore_axis_name="s")

    @functools.partial(
        pl.kernel,
        out_type=jax.ShapeDtypeStruct((NC, npad, fh), F32),
        mesh=mesh,
        scratch_types=[
            pltpu.VMEM((ncht, CH), jnp.int32),
            pltpu.VMEM((ncht, CH), jnp.int32),
            pltpu.VMEM((NB, CH, fh), F32),
            pltpu.VMEM_SHARED((npad, fh), F32),
            [pltpu.SemaphoreType.DMA] * NB,
            [pltpu.SemaphoreType.DMA] * NB,
        ],
        compiler_params=pltpu.CompilerParams(use_tc_tiling_on_sc=False),
    )
    def aggk(g_hbm, src_hbm, dst_hbm, out_hbm, sidx, didx, rows, acc,
             gsem, ssem):
        cid = lax.axis_index("c")
        sid = lax.axis_index("s")
        if edge_split:
            gtab = g_hbm
            pltpu.sync_copy(src_hbm.at[sid, pl.ds(cid * ncht, ncht)], sidx)
            pltpu.sync_copy(dst_hbm.at[sid, pl.ds(cid * ncht, ncht)], didx)
        else:
            gtab = g_hbm.at[cid]
            pltpu.sync_copy(src_hbm.at[sid], sidx)
            pltpu.sync_copy(dst_hbm.at[sid], didx)
        _fill(rows, 0, CH, fh, 0.0)
        for k in range(zr // CH):
            pltpu.sync_copy(rows.at[0], acc.at[pl.ds(sid * zr + k * CH, CH)])
        plsc.subcore_barrier()

        def visit(v, b, swait, fire):
            pltpu.make_async_copy(gtab.at[sidx.at[v]], rows.at[b],
                                  gsem[b]).wait()
            pltpu.async_copy(rows.at[b], acc.at[didx.at[v]], ssem[b],
                             add=True)
            c = (b + FD) % NB
            if swait:
                pltpu.make_async_copy(rows.at[c], acc.at[didx.at[v + FD - NB]],
                                      ssem[c]).wait()
            if fire:
                pltpu.async_copy(gtab.at[sidx.at[v + FD]], rows.at[c],
                                 gsem[c])

        for b in range(FD):
            pltpu.async_copy(gtab.at[sidx.at[b]], rows.at[b], gsem[b])
        for b in range(NB):
            visit(b, b, swait=(b >= NB - FD), fire=True)

        def body(jj, c):
            for b in range(NB):
                visit(jj * NB + b, b, swait=True, fire=True)
            return c

        lax.fori_loop(1, ncht // NB - 1, body, 0)
        for b in range(NB):
            v = ncht - NB + b
            visit(v, b, swait=(v + FD < ncht), fire=(v + FD < ncht))
        for b in range(NB):
            pltpu.make_async_copy(rows.at[b], acc.at[didx.at[ncht - NB + b]],
                                  ssem[b]).wait()

        plsc.subcore_barrier()
        for k in range(zr // CH):
            s = sid * zr + k * CH
            pltpu.sync_copy(acc.at[pl.ds(s, CH)], out_hbm.at[cid, pl.ds(s, CH)])

    return aggk


def _tc_first(x, W1, degp, n_real, npad, blk):
    fin, f1 = W1.shape
    fh = f1 // 2

    def body(d0_ref, d1_ref, x_ref, w_ref, o_ref):
        deg = d0_ref[0, :, 0:1] + d1_ref[0, :, 0:1] + 1.0
        dinv = lax.rsqrt(deg)
        h = jnp.dot(x_ref[...], w_ref[...], preferred_element_type=F32)
        g = h * dinv
        o_ref[0] = g[:, :fh]
        o_ref[1] = g[:, fh:]

    return pl.pallas_call(
        body,
        grid=(npad // blk,),
        in_specs=[
            pl.BlockSpec((1, blk, LANES), lambda i: (0, i, 0)),
            pl.BlockSpec((1, blk, LANES), lambda i: (1, i, 0)),
            pl.BlockSpec((blk, fin), lambda i: (i, 0)),
            pl.BlockSpec((fin, f1), lambda i: (0, 0)),
        ],
        out_specs=pl.BlockSpec((NC, blk, fh), lambda i: (0, i, 0)),
        out_shape=jax.ShapeDtypeStruct((NC, npad, fh), F32),
    )(degp, degp, x, W1)


def _tc_mid(s1, g1, degp, W2, b1, n_real, npad, blk):
    f1, f2 = W2.shape
    fh1 = f1 // 2
    fh2 = f2 // 2

    def body(d0_ref, d1_ref, s_ref, g_ref, w_ref, b_ref, o_ref):
        i = pl.program_id(0)
        deg = d0_ref[0, :, 0:1] + d1_ref[0, :, 0:1] + 1.0
        dinv = lax.rsqrt(deg)
        tot = jnp.concatenate([s_ref[0] + g_ref[0], s_ref[1] + g_ref[1]],
                              axis=1)
        u = jnp.maximum(tot * dinv + b_ref[...], 0.0)
        rows = i * blk + lax.broadcasted_iota(jnp.int32, (blk, 1), 0)
        u = jnp.where(rows < n_real, u, 0.0)
        g = jnp.dot(u, w_ref[...], preferred_element_type=F32) * dinv
        o_ref[0] = g[:, :fh2]
        o_ref[1] = g[:, fh2:]

    return pl.pallas_call(
        body,
        grid=(npad // blk,),
        in_specs=[
            pl.BlockSpec((1, blk, LANES), lambda i: (0, i, 0)),
            pl.BlockSpec((1, blk, LANES), lambda i: (1, i, 0)),
            pl.BlockSpec((NC, blk, fh1), lambda i: (0, i, 0)),
            pl.BlockSpec((NC, blk, fh1), lambda i: (0, i, 0)),
            pl.BlockSpec((f1, f2), lambda i: (0, 0)),
            pl.BlockSpec((1, f1), lambda i: (0, 0)),
        ],
        out_specs=pl.BlockSpec((NC, blk, fh2), lambda i: (0, i, 0)),
        out_shape=jax.ShapeDtypeStruct((NC, npad, fh2), F32),
    )(degp, degp, s1, g1, W2, b1.reshape(1, f1))


def _tc_last(s2, g2, degp, b2, npad, blk):
    f2 = 2 * g2.shape[2]
    fh2 = f2 // 2

    def body(d0_ref, d1_ref, s_ref, g_ref, b_ref, o_ref):
        deg = d0_ref[0, :, 0:1] + d1_ref[0, :, 0:1] + 1.0
        dinv = lax.rsqrt(deg)
        tot = jnp.concatenate([s_ref[0] + g_ref[0], s_ref[1] + g_ref[1]],
                              axis=1)
        o_ref[...] = tot * dinv + b_ref[...]

    return pl.pallas_call(
        body,
        grid=(npad // blk,),
        in_specs=[
            pl.BlockSpec((1, blk, LANES), lambda i: (0, i, 0)),
            pl.BlockSpec((1, blk, LANES), lambda i: (1, i, 0)),
            pl.BlockSpec((NC, blk, fh2), lambda i: (0, i, 0)),
            pl.BlockSpec((NC, blk, fh2), lambda i: (0, i, 0)),
            pl.BlockSpec((1, f2), lambda i: (0, 0)),
        ],
        out_specs=pl.BlockSpec((blk, f2), lambda i: (i, 0)),
        out_shape=jax.ShapeDtypeStruct((npad, f2), F32),
    )(degp, degp, s2, g2, b2.reshape(1, f2))


def kernel(x, edge_index, W1, b1, W2, b2):
    n, fin = x.shape
    e = edge_index.shape[1]
    f1 = W1.shape[1]
    f2 = W2.shape[1]

    npad = -((n + 1) // -(NS * CH)) * (NS * CH)
    ep = -(e // -(NS * CH * 8)) * (NS * CH * 8)
    nch = ep // (NS * CH)

    xp = jnp.zeros((npad, fin), F32).at[:n].set(x)
    pad = jnp.full((ep - e,), n, jnp.int32)
    srcp = jnp.concatenate([edge_index[0], pad]).reshape(NS, nch, CH)
    dstp = jnp.concatenate([edge_index[1], pad]).reshape(NS, nch, CH)

    degp = _make_sc_deg(npad, nch)(dstp)
    g1 = _tc_first(xp, W1, degp, n, npad, 2048)
    s1 = _make_sc_agg(npad, nch, f1 // 2, False)(g1, srcp, dstp)
    g2 = _tc_mid(s1, g1, degp, W2, b1, n, npad, 2048)
    s2 = _make_sc_agg(npad, nch, f2 // 2, False)(g2, srcp, dstp)
    z = _tc_last(s2, g2, degp, b2, npad, 2048)
    return z[:n]


# speedup vs baseline: 1.0715x; 1.0073x over previous
import functools

import jax
import jax.numpy as jnp
from jax import lax
from jax.experimental import pallas as pl
from jax.experimental.pallas import tpu as pltpu
from jax.experimental.pallas import tpu_sc as plsc

NC = 2
NS = 16
NW = NC * NS
CH = 128
LANES = 16

F32 = jnp.float32


def _fill(ref, b, rows, cols, val):
    v16 = jnp.full((LANES,), val, F32)
    groups = cols // LANES

    def body(i, c):
        r = i // groups
        g = i - r * groups
        ref[b, r, pl.ds(g * LANES, LANES)] = v16
        return c

    lax.fori_loop(0, rows * groups, body, 0)


def _make_sc_deg(npad, nch):
    zr = npad // NS
    nch2 = nch // 2
    mesh = plsc.VectorSubcoreMesh(core_axis_name="c", subcore_axis_name="s")

    @functools.partial(
        pl.kernel,
        out_type=jax.ShapeDtypeStruct((NC, npad, LANES), F32),
        mesh=mesh,
        scratch_types=[
            pltpu.VMEM((nch2, CH), jnp.int32),
            pltpu.VMEM((1, CH, LANES), F32),
            pltpu.VMEM_SHARED((npad, LANES), F32),
        ],
        compiler_params=pltpu.CompilerParams(use_tc_tiling_on_sc=False),
    )
    def degk(dst_hbm, out_hbm, didx, ones_rows, acc):
        cid = lax.axis_index("c")
        sid = lax.axis_index("s")
        pltpu.sync_copy(dst_hbm.at[sid, pl.ds(cid * nch2, nch2)], didx)
        _fill(ones_rows, 0, CH, LANES, 0.0)
        for k in range(zr // CH):
            pltpu.sync_copy(ones_rows.at[0],
                            acc.at[pl.ds(sid * zr + k * CH, CH)])
        _fill(ones_rows, 0, CH, LANES, 1.0)
        plsc.subcore_barrier()

        def body(j, c):
            pltpu.sync_copy(ones_rows.at[0], acc.at[didx.at[j]], add=True)
            return c

        lax.fori_loop(0, nch2, body, 0)
        plsc.subcore_barrier()
        for k in range(zr // CH):
            s = sid * zr + k * CH
            pltpu.sync_copy(acc.at[pl.ds(s, CH)], out_hbm.at[cid, pl.ds(s, CH)])

    return degk


def _make_sc_agg(npad, nch, fh, edge_split):
    zr = npad // NS
    NB = 8
    FD = 4
    ncht = nch // 2 if edge_split else nch
    assert ncht % NB == 0 and ncht >= 2 * NB
    mesh = plsc.VectorSubcoreMesh(core_axis_name="c", subcore_axis_name="s")

    @functools.partial(
        pl.kernel,
        out_type=jax.ShapeDtypeStruct((NC, npad, fh), F32),
        mesh=mesh,
        scratch_types=[
            pltpu.VMEM((ncht, CH), jnp.int32),
            pltpu.VMEM((ncht, CH), jnp.int32),
            pltpu.VMEM((NB, CH, fh), F32),
            pltpu.VMEM_SHARED((npad, fh), F32),
            [pltpu.SemaphoreType.DMA] * NB,
            [pltpu.SemaphoreType.DMA] * NB,
        ],
        compiler_params=pltpu.CompilerParams(use_tc_tiling_on_sc=False),
    )
    def aggk(g_hbm, src_hbm, dst_hbm, out_hbm, sidx, didx, rows, acc,
             gsem, ssem):
        cid = lax.axis_index("c")
        sid = lax.axis_index("s")
        if edge_split:
            gtab = g_hbm
            pltpu.sync_copy(src_hbm.at[sid, pl.ds(cid * ncht, ncht)], sidx)
            pltpu.sync_copy(dst_hbm.at[sid, pl.ds(cid * ncht, ncht)], didx)
        else:
            gtab = g_hbm.at[cid]
            pltpu.sync_copy(src_hbm.at[sid], sidx)
            pltpu.sync_copy(dst_hbm.at[sid], didx)
        _fill(rows, 0, CH, fh, 0.0)
        for k in range(zr // CH):
            pltpu.sync_copy(rows.at[0], acc.at[pl.ds(sid * zr + k * CH, CH)])
        plsc.subcore_barrier()

        def visit(v, b, swait, fire):
            pltpu.make_async_copy(gtab.at[sidx.at[v]], rows.at[b],
                                  gsem[b]).wait()
            pltpu.async_copy(rows.at[b], acc.at[didx.at[v]], ssem[b],
                             add=True)
            c = (b + FD) % NB
            if swait:
                pltpu.make_async_copy(rows.at[c], acc.at[didx.at[v + FD - NB]],
                                      ssem[c]).wait()
            if fire:
                pltpu.async_copy(gtab.at[sidx.at[v + FD]], rows.at[c],
                                 gsem[c])

        for b in range(FD):
            pltpu.async_copy(gtab.at[sidx.at[b]], rows.at[b], gsem[b])
        for b in range(NB):
            visit(b, b, swait=(b >= NB - FD), fire=True)

        def body(jj, c):
            for b in range(NB):
                visit(jj * NB + b, b, swait=True, fire=True)
            return c

        lax.fori_loop(1, ncht // NB - 1, body, 0)
        for b in range(NB):
            v = ncht - NB + b
            visit(v, b, swait=(v + FD < ncht), fire=(v + FD < ncht))
        for b in range(NB):
            pltpu.make_async_copy(rows.at[b], acc.at[didx.at[ncht - NB + b]],
                                  ssem[b]).wait()

        plsc.subcore_barrier()
        for k in range(zr // CH):
            s = sid * zr + k * CH
            pltpu.sync_copy(acc.at[pl.ds(s, CH)], out_hbm.at[cid, pl.ds(s, CH)])

    return aggk


def _tc_first(x, W1, degp, n_real, npad, blk):
    fin, f1 = W1.shape
    fh = f1 // 2

    def body(d0_ref, d1_ref, x_ref, w_ref, o_ref):
        deg = d0_ref[0, :, 0:1] + d1_ref[0, :, 0:1] + 1.0
        dinv = lax.rsqrt(deg)
        h = jnp.dot(x_ref[...], w_ref[...], preferred_element_type=F32)
        g = h * dinv
        o_ref[0] = g[:, :fh]
        o_ref[1] = g[:, fh:]

    return pl.pallas_call(
        body,
        grid=(npad // blk,),
        in_specs=[
            pl.BlockSpec((1, blk, LANES), lambda i: (0, i, 0)),
            pl.BlockSpec((1, blk, LANES), lambda i: (1, i, 0)),
            pl.BlockSpec((blk, fin), lambda i: (i, 0)),
            pl.BlockSpec((fin, f1), lambda i: (0, 0)),
        ],
        out_specs=pl.BlockSpec((NC, blk, fh), lambda i: (0, i, 0)),
        out_shape=jax.ShapeDtypeStruct((NC, npad, fh), F32),
    )(degp, degp, x, W1)


def _tc_mid(s1, g1, degp, W2, b1, n_real, npad, blk):
    f1, f2 = W2.shape
    fh1 = f1 // 2
    fh2 = f2 // 2

    def body(d0_ref, d1_ref, s_ref, g_ref, w_ref, b_ref, o_ref):
        i = pl.program_id(0)
        deg = d0_ref[0, :, 0:1] + d1_ref[0, :, 0:1] + 1.0
        dinv = lax.rsqrt(deg)
        tot = jnp.concatenate([s_ref[0] + g_ref[0], s_ref[1] + g_ref[1]],
                              axis=1)
        u = jnp.maximum(tot * dinv + b_ref[...], 0.0)
        rows = i * blk + lax.broadcasted_iota(jnp.int32, (blk, 1), 0)
        u = jnp.where(rows < n_real, u, 0.0)
        g = jnp.dot(u, w_ref[...], preferred_element_type=F32) * dinv
        o_ref[0] = g[:, :fh2]
        o_ref[1] = g[:, fh2:]

    return pl.pallas_call(
        body,
        grid=(npad // blk,),
        in_specs=[
            pl.BlockSpec((1, blk, LANES), lambda i: (0, i, 0)),
            pl.BlockSpec((1, blk, LANES), lambda i: (1, i, 0)),
            pl.BlockSpec((NC, blk, fh1), lambda i: (0, i, 0)),
            pl.BlockSpec((NC, blk, fh1), lambda i: (0, i, 0)),
            pl.BlockSpec((f1, f2), lambda i: (0, 0)),
            pl.BlockSpec((1, f1), lambda i: (0, 0)),
        ],
        out_specs=pl.BlockSpec((NC, blk, fh2), lambda i: (0, i, 0)),
        out_shape=jax.ShapeDtypeStruct((NC, npad, fh2), F32),
    )(degp, degp, s1, g1, W2, b1.reshape(1, f1))


def _tc_last(s2, g2, degp, b2, npad, blk):
    f2 = 2 * g2.shape[2]
    fh2 = f2 // 2

    def body(d0_ref, d1_ref, s_ref, g_ref, b_ref, o_ref):
        deg = d0_ref[0, :, 0:1] + d1_ref[0, :, 0:1] + 1.0
        dinv = lax.rsqrt(deg)
        tot = jnp.concatenate([s_ref[0] + g_ref[0], s_ref[1] + g_ref[1]],
                              axis=1)
        o_ref[...] = tot * dinv + b_ref[...]

    return pl.pallas_call(
        body,
        grid=(npad // blk,),
        in_specs=[
            pl.BlockSpec((1, blk, LANES), lambda i: (0, i, 0)),
            pl.BlockSpec((1, blk, LANES), lambda i: (1, i, 0)),
            pl.BlockSpec((NC, blk, fh2), lambda i: (0, i, 0)),
            pl.BlockSpec((NC, blk, fh2), lambda i: (0, i, 0)),
            pl.BlockSpec((1, f2), lambda i: (0, 0)),
        ],
        out_specs=pl.BlockSpec((blk, f2), lambda i: (i, 0)),
        out_shape=jax.ShapeDtypeStruct((npad, f2), F32),
    )(degp, degp, s2, g2, b2.reshape(1, f2))


def kernel(x, edge_index, W1, b1, W2, b2):
    n, fin = x.shape
    e = edge_index.shape[1]
    f1 = W1.shape[1]
    f2 = W2.shape[1]

    npad = -((n + 1) // -(NS * CH)) * (NS * CH)
    ep = -(e // -(NS * CH * 8)) * (NS * CH * 8)
    nch = ep // (NS * CH)

    xp = jnp.zeros((npad, fin), F32).at[:n].set(x)
    pad = jnp.full((ep - e,), n, jnp.int32)
    srcp = jnp.concatenate([edge_index[0], pad]).reshape(NS, nch, CH)
    dstp = jnp.concatenate([edge_index[1], pad]).reshape(NS, nch, CH)

    degp = _make_sc_deg(npad, nch)(dstp)
    g1 = _tc_first(xp, W1, degp, n, npad, 2560)
    s1 = _make_sc_agg(npad, nch, f1 // 2, False)(g1, srcp, dstp)
    g2 = _tc_mid(s1, g1, degp, W2, b1, n, npad, 2560)
    s2 = _make_sc_agg(npad, nch, f2 // 2, False)(g2, srcp, dstp)
    z = _tc_last(s2, g2, degp, b2, npad, 2560)
    return z[:n]
